# baseline (speedup 1.0000x reference)
import functools

import jax
import jax.numpy as jnp
from jax import lax
from jax.experimental import pallas as pl
from jax.experimental.pallas import tpu as pltpu
from jax.experimental.pallas import tpu_sc as plsc

NUM_ET = 4
N_NODES = 100000
N_EDGES = 150000
IN_DIM = 128

NW = 32
CB = 80
NBLK = N_EDGES // CB
NG = CB // 16


def _sc_kernel_body(z0, z1, z2, z3, s0, d0, s1, d1, s2, d2, s3, d3,
                    w_hbm, sig_hbm, out_hbm,
                    idx_s, idx_d, rows_s, rows_d, out_v, w_v, sig_v, sem):
    cid = lax.axis_index("c")
    sid = lax.axis_index("s")
    wid = sid * 2 + cid

    pltpu.sync_copy(w_hbm, w_v)
    pltpu.sync_copy(sig_hbm, sig_v)
    sig = sig_v[...]
    lane = lax.iota(jnp.int32, 16)

    nit = (NBLK - wid + (NW - 1)) // NW

    for et, (z_t, s_t, d_t) in enumerate(
            ((z0, s0, d0), (z1, s1, d1), (z2, s2, d2), (z3, s3, d3))):
        def blk(i, carry, z_t=z_t, s_t=s_t, d_t=d_t, et=et):
            b = wid + i * NW
            base = b * CB
            pltpu.sync_copy(s_t.at[pl.ds(base, CB)], idx_s)
            pltpu.sync_copy(d_t.at[pl.ds(base, CB)], idx_d)
            c1 = pltpu.make_async_copy(z_t.at[idx_s], rows_s, sem)
            c1.start()
            c2 = pltpu.make_async_copy(z_t.at[idx_d], rows_d, sem)
            c2.start()
            c1.wait()
            c2.wait()
            for g in range(NG):
                rid = lane + (g * 16)

                def dbody(dg, acc, rid=rid, et=et):
                    wv = w_v[pl.ds(et * IN_DIM + dg * 16, 16)]
                    colb = jnp.full((16,), dg * 16, jnp.int32)
                    for j in range(16):
                        col = colb + j
                        sv = plsc.load_gather(rows_s, [rid, col])
                        tv = plsc.load_gather(rows_d, [rid, col])
                        acc = acc + sv * tv * wv[j]
                    return acc

                acc = lax.fori_loop(0, IN_DIM // 16, dbody,
                                    jnp.zeros((16,), jnp.float32))
                sgm = 1.0 / (1.0 + jnp.exp(-acc))
                out_v[pl.ds(g * 16, 16)] = acc + sig * (sgm - acc)
            pltpu.sync_copy(out_v, out_hbm.at[pl.ds(et * N_EDGES + base, CB)])
            return carry
        lax.fori_loop(0, nit, blk, 0)


@jax.jit
def _decode_all(z, edge_index, weight, sig_f32):
    mesh = plsc.VectorSubcoreMesh(core_axis_name="c", subcore_axis_name="s")
    run = functools.partial(
        pl.kernel,
        mesh=mesh,
        out_type=jax.ShapeDtypeStruct((NUM_ET * N_EDGES,), jnp.float32),
        scratch_types=[
            pltpu.VMEM((CB,), jnp.int32),
            pltpu.VMEM((CB,), jnp.int32),
            pltpu.VMEM((CB, IN_DIM), jnp.float32),
            pltpu.VMEM((CB, IN_DIM), jnp.float32),
            pltpu.VMEM((CB,), jnp.float32),
            pltpu.VMEM((NUM_ET * IN_DIM,), jnp.float32),
            pltpu.VMEM((16,), jnp.float32),
            pltpu.SemaphoreType.DMA,
        ],
        compiler_params=pltpu.CompilerParams(needs_layout_passes=False),
    )(_sc_kernel_body)
    sig_vec = jnp.full((16,), 1.0, jnp.float32) * sig_f32
    return run(z[0], z[1], z[2], z[3],
               edge_index[0, 0], edge_index[0, 1],
               edge_index[1, 0], edge_index[1, 1],
               edge_index[2, 0], edge_index[2, 1],
               edge_index[3, 0], edge_index[3, 1],
               weight.reshape(-1), sig_vec)


def kernel(z, edge_index, weight, sigmoid):
    sig_f32 = jnp.asarray(sigmoid, jnp.float32)
    out = _decode_all(z, edge_index, weight, sig_f32)
    per_et = tuple(out[et * N_EDGES:(et + 1) * N_EDGES] for et in range(NUM_ET))
    return (per_et, out)


# speedup vs baseline: 1.0472x; 1.0472x over previous
import functools

import jax
import jax.numpy as jnp
from jax import lax
from jax.experimental import pallas as pl
from jax.experimental.pallas import tpu as pltpu
from jax.experimental.pallas import tpu_sc as plsc

NUM_ET = 4
N_NODES = 100000
N_EDGES = 150000
IN_DIM = 128

NW = 32
CB = 80
NBLK = N_EDGES // CB
NG = CB // 16


def _sc_kernel_body(z_all, src_all, dst_all, w_hbm, sig_hbm, out_hbm,
                    idx_s, idx_d, rows_s, rows_d, out_v, w_v, sig_v, sem):
    cid = lax.axis_index("c")
    sid = lax.axis_index("s")
    wid = sid * 2 + cid

    pltpu.sync_copy(w_hbm, w_v)
    pltpu.sync_copy(sig_hbm, sig_v)
    sig = sig_v[...]
    lane = lax.iota(jnp.int32, 16)

    nit = (NBLK - wid + (NW - 1)) // NW

    def etloop(et, carry):
        w_vecs = [w_v[pl.ds(et * IN_DIM + dg * 16, 16)]
                  for dg in range(IN_DIM // 16)]
        row_off = jnp.zeros((16,), jnp.int32) + et * N_NODES

        def blk(i, carry2, w_vecs=w_vecs):
            b = wid + i * NW
            ebase = et * N_EDGES + b * CB
            pltpu.sync_copy(src_all.at[pl.ds(ebase, CB)], idx_s)
            pltpu.sync_copy(dst_all.at[pl.ds(ebase, CB)], idx_d)
            for q in range(CB // 16):
                idx_s[pl.ds(q * 16, 16)] = idx_s[pl.ds(q * 16, 16)] + row_off
                idx_d[pl.ds(q * 16, 16)] = idx_d[pl.ds(q * 16, 16)] + row_off
            c1 = pltpu.make_async_copy(z_all.at[idx_s], rows_s, sem)
            c1.start()
            c2 = pltpu.make_async_copy(z_all.at[idx_d], rows_d, sem)
            c2.start()
            c1.wait()
            c2.wait()

            zero = jnp.zeros((16,), jnp.float32)

            @plsc.parallel_loop(0, NG)
            def gloop(g):
                rid = lane + g * 16

                @plsc.parallel_loop(0, IN_DIM // 16, unroll=2,
                                    carry=(zero, zero, zero, zero))
                def dloop(dg, accs):
                    a0, a1, a2, a3 = accs
                    wv = w_v[pl.ds(et * IN_DIM + dg * 16, 16)]
                    colb = jnp.zeros((16,), jnp.int32) + dg * 16
                    for j in range(16):
                        col = colb + j
                        sv = plsc.load_gather(rows_s, [rid, col])
                        tv = plsc.load_gather(rows_d, [rid, col])
                        p = sv * tv * wv[j]
                        if j % 4 == 0:
                            a0 = a0 + p
                        elif j % 4 == 1:
                            a1 = a1 + p
                        elif j % 4 == 2:
                            a2 = a2 + p
                        else:
                            a3 = a3 + p
                    return (a0, a1, a2, a3)

                a0, a1, a2, a3 = dloop
                acc = (a0 + a1) + (a2 + a3)
                sgm = 1.0 / (1.0 + jnp.exp(-acc))
                out_v[pl.ds(g * 16, 16)] = acc + sig * (sgm - acc)
            pltpu.sync_copy(out_v, out_hbm.at[pl.ds(ebase, CB)])
            return carry2

        lax.fori_loop(0, nit, blk, 0)
        return carry

    lax.fori_loop(0, NUM_ET, etloop, 0)


@jax.jit
def _decode_all(z, edge_index, weight, sig_f32):
    mesh = plsc.VectorSubcoreMesh(core_axis_name="c", subcore_axis_name="s")
    run = functools.partial(
        pl.kernel,
        mesh=mesh,
        out_type=jax.ShapeDtypeStruct((NUM_ET * N_EDGES,), jnp.float32),
        scratch_types=[
            pltpu.VMEM((CB,), jnp.int32),
            pltpu.VMEM((CB,), jnp.int32),
            pltpu.VMEM((CB, IN_DIM), jnp.float32),
            pltpu.VMEM((CB, IN_DIM), jnp.float32),
            pltpu.VMEM((CB,), jnp.float32),
            pltpu.VMEM((NUM_ET * IN_DIM,), jnp.float32),
            pltpu.VMEM((16,), jnp.float32),
            pltpu.SemaphoreType.DMA,
        ],
        compiler_params=pltpu.CompilerParams(needs_layout_passes=False),
    )(_sc_kernel_body)
    sig_vec = jnp.full((16,), 1.0, jnp.float32) * sig_f32
    return run(z.reshape(NUM_ET * N_NODES, IN_DIM),
               edge_index[:, 0, :].reshape(-1),
               edge_index[:, 1, :].reshape(-1),
               weight.reshape(-1), sig_vec)


def kernel(z, edge_index, weight, sigmoid):
    sig_f32 = jnp.asarray(sigmoid, jnp.float32)
    out = _decode_all(z, edge_index, weight, sig_f32)
    per_et = tuple(out[et * N_EDGES:(et + 1) * N_EDGES] for et in range(NUM_ET))
    return (per_et, out)
